# baseline (speedup 1.0000x reference)
import jax
import jax.numpy as jnp
from jax.experimental import pallas as pl

_B, _S, _D, _E, _K = 4, 4096, 2048, 16, 2
_M = _B * _S
_BM = 2048
_SPB = _S // _BM


def _router_body(x_ref, w_ref, lt_ref, wt_ref, it_ref):
    logits_t = jax.lax.dot_general(
        w_ref[...],
        x_ref[...],
        ((( 1,), (1,)), ((), ())),
        preferred_element_type=jnp.float32,
    )
    lt_ref[...] = logits_t[None]

    m1 = jnp.max(logits_t, axis=0)
    i1 = jnp.argmax(logits_t, axis=0).astype(jnp.int32)
    row = jax.lax.broadcasted_iota(jnp.int32, logits_t.shape, 0)
    masked = jnp.where(row == i1[None, :], -jnp.inf, logits_t)
    m2 = jnp.max(masked, axis=0)
    i2 = jnp.argmax(masked, axis=0).astype(jnp.int32)

    e2 = jnp.exp(m2 - m1)
    denom = 1.0 + e2
    wt_ref[...] = jnp.stack([1.0 / denom, e2 / denom], axis=0)[None]
    it_ref[...] = jnp.stack([i1, i2], axis=0)[None]


@jax.jit
def kernel(x, W):
    xm = x.reshape(_M, _D)

    lt, wt, it = pl.pallas_call(
        _router_body,
        grid=(_M // _BM,),
        in_specs=[
            pl.BlockSpec((_BM, _D), lambda i: (i, 0)),
            pl.BlockSpec((_E, _D), lambda i: (0, 0)),
        ],
        out_specs=[
            pl.BlockSpec((1, _E, _BM), lambda i: (i // _SPB, 0, i % _SPB)),
            pl.BlockSpec((1, _K, _BM), lambda i: (i // _SPB, 0, i % _SPB)),
            pl.BlockSpec((1, _K, _BM), lambda i: (i // _SPB, 0, i % _SPB)),
        ],
        out_shape=[
            jax.ShapeDtypeStruct((_B, _E, _S), jnp.float32),
            jax.ShapeDtypeStruct((_B, _K, _S), jnp.float32),
            jax.ShapeDtypeStruct((_B, _K, _S), jnp.int32),
        ],
    )(xm, W)

    return (
        wt.transpose(0, 2, 1),
        it.transpose(0, 2, 1),
        lt.transpose(0, 2, 1),
    )
